# baseline (speedup 1.0000x reference)
import jax
import jax.numpy as jnp
from jax.experimental import pallas as pl

N = 50000


def _gat_conv(x, edge_index, W, a_src, a_dst, bias, num_nodes):
    loop = jnp.arange(num_nodes, dtype=edge_index.dtype)
    src = jnp.concatenate([edge_index[0], loop])
    dst = jnp.concatenate([edge_index[1], loop])
    h = x @ W
    alpha_s = h @ a_src
    alpha_d = h @ a_dst
    e = alpha_s[src] + alpha_d[dst]
    e = jax.nn.leaky_relu(e, 0.2)
    e_max = jax.ops.segment_max(e, dst, num_segments=num_nodes)
    e_max = jnp.where(jnp.isfinite(e_max), e_max, 0.0)
    e_exp = jnp.exp(e - e_max[dst])
    denom = jax.ops.segment_sum(e_exp, dst, num_segments=num_nodes)
    alpha = e_exp / (denom[dst] + 1e-16)
    out = jax.ops.segment_sum(h[src] * alpha[:, None], dst, num_segments=num_nodes)
    return out + bias


def _copy_body(x_ref, o_ref):
    o_ref[...] = x_ref[...]


def kernel(x, edge_index, W1, a1_src, a1_dst, b1, W2, a2_src, a2_dst, b2):
    h = _gat_conv(x, edge_index, W1, a1_src, a1_dst, b1, N)
    h = jax.nn.elu(h)
    out = _gat_conv(h, edge_index, W2, a2_src, a2_dst, b2, N)
    out = pl.pallas_call(
        _copy_body,
        out_shape=jax.ShapeDtypeStruct(out.shape, out.dtype),
        grid=(25,),
        in_specs=[pl.BlockSpec((2000, 16), lambda i: (i, 0))],
        out_specs=pl.BlockSpec((2000, 16), lambda i: (i, 0)),
    )(out)
    return out


# speedup vs baseline: 30.6724x; 30.6724x over previous
import functools

import jax
import jax.numpy as jnp
from jax import lax
from jax.experimental import pallas as pl
from jax.experimental.pallas import tpu as pltpu
from jax.experimental.pallas import tpu_sc as plsc

N = 50000
E = 1600000
F = 16
BN = 2000
GRID = N // BN
EB = 128
NCHUNK = E // EB
RC = 200
NRC = N // RC
NTILES = 32


def _prep_body(x_ref, w_ref, as_ref, ad_ref, h_out, asv_out, adv_out):
    h = jnp.dot(x_ref[...], w_ref[...], preferred_element_type=jnp.float32)
    h_out[...] = h
    asv_out[...] = jnp.dot(h, as_ref[...])
    adv_out[...] = jnp.dot(h, ad_ref[...])


def _tc_prep(x, W, a_src, a_dst):
    in_dim = x.shape[1]
    return pl.pallas_call(
        _prep_body,
        grid=(GRID,),
        in_specs=[
            pl.BlockSpec((BN, in_dim), lambda i: (i, 0)),
            pl.BlockSpec((in_dim, F), lambda i: (0, 0)),
            pl.BlockSpec((F, 1), lambda i: (0, 0)),
            pl.BlockSpec((F, 1), lambda i: (0, 0)),
        ],
        out_specs=[
            pl.BlockSpec((BN, F), lambda i: (i, 0)),
            pl.BlockSpec((BN, 1), lambda i: (i, 0)),
            pl.BlockSpec((BN, 1), lambda i: (i, 0)),
        ],
        out_shape=[
            jax.ShapeDtypeStruct((N, F), jnp.float32),
            jax.ShapeDtypeStruct((N, 1), jnp.float32),
            jax.ShapeDtypeStruct((N, 1), jnp.float32),
        ],
    )(x, W, a_src.reshape(F, 1), a_dst.reshape(F, 1))


def _mid_body(acch_ref, accw_ref, h_ref, b_ref, w2_ref, as_ref, ad_ref,
              h2_out, asv_out, adv_out):
    num = acch_ref[0] + acch_ref[1] + h_ref[...]
    den = accw_ref[0] + accw_ref[1] + 1.0
    h1 = num / den + b_ref[...]
    z = jnp.where(h1 > 0.0, h1, jnp.exp(jnp.minimum(h1, 0.0)) - 1.0)
    h2 = jnp.dot(z, w2_ref[...], preferred_element_type=jnp.float32)
    a_s = jnp.dot(h2, as_ref[...])
    a_d = jnp.dot(h2, ad_ref[...])
    h2_out[...] = h2
    asv_out[...] = a_s
    adv_out[...] = a_d


def _tc_mid(acc_h, acc_w, h, b, W2, a_src, a_dst):
    return pl.pallas_call(
        _mid_body,
        grid=(GRID,),
        in_specs=[
            pl.BlockSpec((2, BN, F), lambda i: (0, i, 0)),
            pl.BlockSpec((2, BN, 1), lambda i: (0, i, 0)),
            pl.BlockSpec((BN, F), lambda i: (i, 0)),
            pl.BlockSpec((1, F), lambda i: (0, 0)),
            pl.BlockSpec((F, F), lambda i: (0, 0)),
            pl.BlockSpec((F, 1), lambda i: (0, 0)),
            pl.BlockSpec((F, 1), lambda i: (0, 0)),
        ],
        out_specs=[
            pl.BlockSpec((BN, F), lambda i: (i, 0)),
            pl.BlockSpec((BN, 1), lambda i: (i, 0)),
            pl.BlockSpec((BN, 1), lambda i: (i, 0)),
        ],
        out_shape=[
            jax.ShapeDtypeStruct((N, F), jnp.float32),
            jax.ShapeDtypeStruct((N, 1), jnp.float32),
            jax.ShapeDtypeStruct((N, 1), jnp.float32),
        ],
    )(acc_h, acc_w.reshape(2, N, 1), h, b.reshape(1, F), W2,
      a_src.reshape(F, 1), a_dst.reshape(F, 1))


def _fin_body(acch_ref, accw_ref, h_ref, b_ref, out_ref):
    num = acch_ref[0] + acch_ref[1] + h_ref[...]
    den = accw_ref[0] + accw_ref[1] + 1.0
    out_ref[...] = num / den + b_ref[...]


def _tc_fin(acc_h, acc_w, h, b):
    return pl.pallas_call(
        _fin_body,
        grid=(GRID,),
        in_specs=[
            pl.BlockSpec((2, BN, F), lambda i: (0, i, 0)),
            pl.BlockSpec((2, BN, 1), lambda i: (0, i, 0)),
            pl.BlockSpec((BN, F), lambda i: (i, 0)),
            pl.BlockSpec((1, F), lambda i: (0, 0)),
        ],
        out_specs=pl.BlockSpec((BN, F), lambda i: (i, 0)),
        out_shape=jax.ShapeDtypeStruct((N, F), jnp.float32),
    )(acc_h, acc_w.reshape(2, N, 1), h, b.reshape(1, F))


def _edge_body(src_hbm, dst_hbm, h_hbm, as_hbm, ad_hbm,
               zh_hbm, zw_hbm,
               acch_out, accw_out,
               src_i, dst_i, asg, adg, asdg, hg, hw,
               stage_h, stage_s,
               acch_s, accw_s,
               sem):
    c = lax.axis_index("c")
    s = lax.axis_index("s")
    wid = s * 2 + c

    nq = (NRC - s + 15) // 16
    pltpu.sync_copy(zh_hbm, stage_h)
    pltpu.sync_copy(zw_hbm, stage_s)

    def init_body(t, carry):
        r0 = (s + t * 16) * RC
        pltpu.sync_copy(stage_h, acch_s.at[pl.ds(r0, RC)])
        pltpu.sync_copy(stage_s, accw_s.at[pl.ds(r0, RC)])
        return carry

    lax.fori_loop(0, nq, init_body, 0)
    plsc.subcore_barrier()

    nk = (NCHUNK - wid + NTILES - 1) // NTILES

    def edge_chunk(t, carry):
        off = (wid + t * NTILES) * EB
        pltpu.sync_copy(src_hbm.at[pl.ds(off, EB)], src_i)
        pltpu.sync_copy(dst_hbm.at[pl.ds(off, EB)], dst_i)
        pltpu.sync_copy(h_hbm.at[src_i], hg)
        pltpu.sync_copy(as_hbm.at[src_i], asg)
        pltpu.sync_copy(ad_hbm.at[dst_i], adg)
        pltpu.sync_copy(as_hbm.at[dst_i], asdg)
        for j in range(EB // 16):
            sl = pl.ds(j * 16, 16)
            ad16 = adg[sl]
            t0 = asg[sl] + ad16
            e = jnp.maximum(t0, 0.2 * t0)
            td = asdg[sl] + ad16
            cg = jnp.maximum(td, 0.2 * td)
            w = jnp.exp(e - cg)
            asg[sl] = w
            for k in range(16):
                i = j * 16 + k
                hw[i, :] = hg[i, :] * w[k]
        pltpu.sync_copy(hw, acch_s.at[dst_i], add=True)
        pltpu.sync_copy(asg, accw_s.at[dst_i], add=True)
        return carry

    lax.fori_loop(0, nk, edge_chunk, 0)
    plsc.subcore_barrier()

    def wb_body(t, carry):
        r0 = (s + t * 16) * RC
        pltpu.sync_copy(acch_s.at[pl.ds(r0, RC)], stage_h)
        pltpu.sync_copy(stage_h, acch_out.at[c, pl.ds(r0, RC)])
        pltpu.sync_copy(accw_s.at[pl.ds(r0, RC)], stage_s)
        pltpu.sync_copy(stage_s, accw_out.at[pl.ds(c * N + r0, RC)])
        return carry

    lax.fori_loop(0, nq, wb_body, 0)


def _edge_pass(src, dst, h, asv, adv, zh, zw):
    mesh = plsc.VectorSubcoreMesh(core_axis_name="c", subcore_axis_name="s")
    f = functools.partial(
        pl.kernel,
        mesh=mesh,
        compiler_params=pltpu.CompilerParams(use_tc_tiling_on_sc=False),
        out_type=[
            jax.ShapeDtypeStruct((2, N, F), jnp.float32),
            jax.ShapeDtypeStruct((2 * N,), jnp.float32),
        ],
        scratch_types=[
            pltpu.VMEM((EB,), jnp.int32),
            pltpu.VMEM((EB,), jnp.int32),
            pltpu.VMEM((EB,), jnp.float32),
            pltpu.VMEM((EB,), jnp.float32),
            pltpu.VMEM((EB,), jnp.float32),
            pltpu.VMEM((EB, F), jnp.float32),
            pltpu.VMEM((EB, F), jnp.float32),
            pltpu.VMEM((RC, F), jnp.float32),
            pltpu.VMEM((RC,), jnp.float32),
            pltpu.VMEM_SHARED((N, F), jnp.float32),
            pltpu.VMEM_SHARED((N,), jnp.float32),
            pltpu.SemaphoreType.DMA,
        ],
    )(_edge_body)
    return f(src, dst, h, asv, adv, zh, zw)


def kernel(x, edge_index, W1, a1_src, a1_dst, b1, W2, a2_src, a2_dst, b2):
    src = edge_index[0].astype(jnp.int32)
    dst = edge_index[1].astype(jnp.int32)
    zh = jnp.zeros((RC, F), jnp.float32)
    zw = jnp.zeros((RC,), jnp.float32)

    h1, as1, ad1 = _tc_prep(x, W1, a1_src, a1_dst)
    acch1, accw1 = _edge_pass(src, dst, h1,
                              as1.reshape(N), ad1.reshape(N), zh, zw)
    h2, as2, ad2 = _tc_mid(acch1, accw1, h1, b1, W2, a2_src, a2_dst)
    acch2, accw2 = _edge_pass(src, dst, h2,
                              as2.reshape(N), ad2.reshape(N), zh, zw)
    return _tc_fin(acch2, accw2, h2, b2)


# speedup vs baseline: 77.0118x; 2.5108x over previous
import functools

import jax
import jax.numpy as jnp
from jax import lax
from jax.experimental import pallas as pl
from jax.experimental.pallas import tpu as pltpu
from jax.experimental.pallas import tpu_sc as plsc

N = 50000
E = 1600000
F = 16
BN = 2000
GRID = N // BN
EB = 128
NCHUNK = E // EB
G = 4
NSUP = NCHUNK // G
RC = 200
NRC = N // RC
NTILES = 32


def _prep_body(x_ref, w_ref, as_ref, ad_ref, h_out, asv_out, adv_out):
    h = jnp.dot(x_ref[...], w_ref[...], preferred_element_type=jnp.float32)
    h_out[...] = h
    asv_out[...] = jnp.dot(h, as_ref[...])
    adv_out[...] = jnp.dot(h, ad_ref[...])


def _tc_prep(x, W, a_src, a_dst):
    in_dim = x.shape[1]
    return pl.pallas_call(
        _prep_body,
        grid=(GRID,),
        in_specs=[
            pl.BlockSpec((BN, in_dim), lambda i: (i, 0)),
            pl.BlockSpec((in_dim, F), lambda i: (0, 0)),
            pl.BlockSpec((F, 1), lambda i: (0, 0)),
            pl.BlockSpec((F, 1), lambda i: (0, 0)),
        ],
        out_specs=[
            pl.BlockSpec((BN, F), lambda i: (i, 0)),
            pl.BlockSpec((BN, 1), lambda i: (i, 0)),
            pl.BlockSpec((BN, 1), lambda i: (i, 0)),
        ],
        out_shape=[
            jax.ShapeDtypeStruct((N, F), jnp.float32),
            jax.ShapeDtypeStruct((N, 1), jnp.float32),
            jax.ShapeDtypeStruct((N, 1), jnp.float32),
        ],
    )(x, W, a_src.reshape(F, 1), a_dst.reshape(F, 1))


def _mid_body(acch_ref, accw_ref, h_ref, b_ref, w2_ref, as_ref, ad_ref,
              h2_out, asv_out, adv_out):
    num = acch_ref[0] + acch_ref[1] + h_ref[...]
    den = accw_ref[0] + accw_ref[1] + 1.0
    h1 = num / den + b_ref[...]
    z = jnp.where(h1 > 0.0, h1, jnp.exp(jnp.minimum(h1, 0.0)) - 1.0)
    h2 = jnp.dot(z, w2_ref[...], preferred_element_type=jnp.float32)
    a_s = jnp.dot(h2, as_ref[...])
    a_d = jnp.dot(h2, ad_ref[...])
    h2_out[...] = h2
    asv_out[...] = a_s
    adv_out[...] = a_d


def _tc_mid(acc_h, acc_w, h, b, W2, a_src, a_dst):
    return pl.pallas_call(
        _mid_body,
        grid=(GRID,),
        in_specs=[
            pl.BlockSpec((2, BN, F), lambda i: (0, i, 0)),
            pl.BlockSpec((2, BN, 1), lambda i: (0, i, 0)),
            pl.BlockSpec((BN, F), lambda i: (i, 0)),
            pl.BlockSpec((1, F), lambda i: (0, 0)),
            pl.BlockSpec((F, F), lambda i: (0, 0)),
            pl.BlockSpec((F, 1), lambda i: (0, 0)),
            pl.BlockSpec((F, 1), lambda i: (0, 0)),
        ],
        out_specs=[
            pl.BlockSpec((BN, F), lambda i: (i, 0)),
            pl.BlockSpec((BN, 1), lambda i: (i, 0)),
            pl.BlockSpec((BN, 1), lambda i: (i, 0)),
        ],
        out_shape=[
            jax.ShapeDtypeStruct((N, F), jnp.float32),
            jax.ShapeDtypeStruct((N, 1), jnp.float32),
            jax.ShapeDtypeStruct((N, 1), jnp.float32),
        ],
    )(acc_h, acc_w.reshape(2, N, 1), h, b.reshape(1, F), W2,
      a_src.reshape(F, 1), a_dst.reshape(F, 1))


def _fin_body(acch_ref, accw_ref, h_ref, b_ref, out_ref):
    num = acch_ref[0] + acch_ref[1] + h_ref[...]
    den = accw_ref[0] + accw_ref[1] + 1.0
    out_ref[...] = num / den + b_ref[...]


def _tc_fin(acc_h, acc_w, h, b):
    return pl.pallas_call(
        _fin_body,
        grid=(GRID,),
        in_specs=[
            pl.BlockSpec((2, BN, F), lambda i: (0, i, 0)),
            pl.BlockSpec((2, BN, 1), lambda i: (0, i, 0)),
            pl.BlockSpec((BN, F), lambda i: (i, 0)),
            pl.BlockSpec((1, F), lambda i: (0, 0)),
        ],
        out_specs=pl.BlockSpec((BN, F), lambda i: (i, 0)),
        out_shape=jax.ShapeDtypeStruct((N, F), jnp.float32),
    )(acc_h, acc_w.reshape(2, N, 1), h, b.reshape(1, F))


def _edge_body(src_hbm, dst_hbm, h_hbm, as_hbm, ad_hbm,
               zh_hbm, zw_hbm,
               acch_out, accw_out,
               src_i, dst_i, asg, adg, asdg, hg, hw,
               stage_h, stage_s,
               acch_s, accw_s,
               sem):
    c = lax.axis_index("c")
    s = lax.axis_index("s")
    wid = s * 2 + c

    nq = (NRC - s + 15) // 16
    pltpu.sync_copy(zh_hbm, stage_h)
    pltpu.sync_copy(zw_hbm, stage_s)

    def init_body(t, carry):
        r0 = (s + t * 16) * RC
        pltpu.sync_copy(stage_h, acch_s.at[pl.ds(r0, RC)])
        pltpu.sync_copy(stage_s, accw_s.at[pl.ds(r0, RC)])
        return carry

    lax.fori_loop(0, nq, init_body, 0)
    plsc.subcore_barrier()

    nk = (NSUP - wid + NTILES - 1) // NTILES

    def edge_chunk(t, carry):
        r0 = (wid + t * NTILES) * G
        pltpu.sync_copy(src_hbm.at[pl.ds(r0, G)], src_i)
        pltpu.sync_copy(dst_hbm.at[pl.ds(r0, G)], dst_i)
        cps = []
        for j in range(G):
            sl = pl.ds(j * EB, EB)
            cps.append(pltpu.async_copy(h_hbm.at[src_i.at[j]], hg.at[sl], sem))
            cps.append(pltpu.async_copy(as_hbm.at[src_i.at[j]], asg.at[sl], sem))
            cps.append(pltpu.async_copy(ad_hbm.at[dst_i.at[j]], adg.at[sl], sem))
            cps.append(pltpu.async_copy(as_hbm.at[dst_i.at[j]], asdg.at[sl], sem))
        for cp in cps:
            cp.wait()
        for j in range(G * EB // 16):
            sl = pl.ds(j * 16, 16)
            ad16 = adg[sl]
            t0 = asg[sl] + ad16
            e = jnp.maximum(t0, 0.2 * t0)
            td = asdg[sl] + ad16
            cg = jnp.maximum(td, 0.2 * td)
            w = jnp.exp(e - cg)
            asg[sl] = w
            for k in range(16):
                i = j * 16 + k
                hw[i, :] = hg[i, :] * w[k]
        cps = []
        for j in range(G):
            sl = pl.ds(j * EB, EB)
            cps.append(pltpu.async_copy(hw.at[sl], acch_s.at[dst_i.at[j]], sem,
                                        add=True))
            cps.append(pltpu.async_copy(asg.at[sl], accw_s.at[dst_i.at[j]], sem,
                                        add=True))
        for cp in cps:
            cp.wait()
        return carry

    lax.fori_loop(0, nk, edge_chunk, 0)
    plsc.subcore_barrier()

    def wb_body(t, carry):
        r0 = (s + t * 16) * RC
        pltpu.sync_copy(acch_s.at[pl.ds(r0, RC)], stage_h)
        pltpu.sync_copy(stage_h, acch_out.at[c, pl.ds(r0, RC)])
        pltpu.sync_copy(accw_s.at[pl.ds(r0, RC)], stage_s)
        pltpu.sync_copy(stage_s, accw_out.at[pl.ds(c * N + r0, RC)])
        return carry

    lax.fori_loop(0, nq, wb_body, 0)


def _edge_pass(src, dst, h, asv, adv, zh, zw):
    mesh = plsc.VectorSubcoreMesh(core_axis_name="c", subcore_axis_name="s")
    f = functools.partial(
        pl.kernel,
        mesh=mesh,
        compiler_params=pltpu.CompilerParams(use_tc_tiling_on_sc=False),
        out_type=[
            jax.ShapeDtypeStruct((2, N, F), jnp.float32),
            jax.ShapeDtypeStruct((2 * N,), jnp.float32),
        ],
        scratch_types=[
            pltpu.VMEM((G, EB), jnp.int32),
            pltpu.VMEM((G, EB), jnp.int32),
            pltpu.VMEM((G * EB,), jnp.float32),
            pltpu.VMEM((G * EB,), jnp.float32),
            pltpu.VMEM((G * EB,), jnp.float32),
            pltpu.VMEM((G * EB, F), jnp.float32),
            pltpu.VMEM((G * EB, F), jnp.float32),
            pltpu.VMEM((RC, F), jnp.float32),
            pltpu.VMEM((RC,), jnp.float32),
            pltpu.VMEM_SHARED((N, F), jnp.float32),
            pltpu.VMEM_SHARED((N,), jnp.float32),
            pltpu.SemaphoreType.DMA,
        ],
    )(_edge_body)
    return f(src, dst, h, asv, adv, zh, zw)


def kernel(x, edge_index, W1, a1_src, a1_dst, b1, W2, a2_src, a2_dst, b2):
    src = edge_index[0].astype(jnp.int32).reshape(NCHUNK, EB)
    dst = edge_index[1].astype(jnp.int32).reshape(NCHUNK, EB)
    zh = jnp.zeros((RC, F), jnp.float32)
    zw = jnp.zeros((RC,), jnp.float32)

    h1, as1, ad1 = _tc_prep(x, W1, a1_src, a1_dst)
    acch1, accw1 = _edge_pass(src, dst, h1,
                              as1.reshape(N), ad1.reshape(N), zh, zw)
    h2, as2, ad2 = _tc_mid(acch1, accw1, h1, b1, W2, a2_src, a2_dst)
    acch2, accw2 = _edge_pass(src, dst, h2,
                              as2.reshape(N), ad2.reshape(N), zh, zw)
    return _tc_fin(acch2, accw2, h2, b2)


# speedup vs baseline: 90.6741x; 1.1774x over previous
import functools

import jax
import jax.numpy as jnp
from jax import lax
from jax.experimental import pallas as pl
from jax.experimental.pallas import tpu as pltpu
from jax.experimental.pallas import tpu_sc as plsc

N = 50000
E = 1600000
F = 16
BN = 2000
GRID = N // BN
EB = 128
NCHUNK = E // EB
G = 4
NSUP = NCHUNK // G
RC = 200
NRC = N // RC
NTILES = 32


def _prep_body(x_ref, w_ref, as_ref, ad_ref, h_out, asv_out, adv_out):
    h = jnp.dot(x_ref[...], w_ref[...], preferred_element_type=jnp.float32)
    h_out[...] = h
    asv_out[...] = jnp.dot(h, as_ref[...])
    adv_out[...] = jnp.dot(h, ad_ref[...])


def _tc_prep(x, W, a_src, a_dst):
    in_dim = x.shape[1]
    return pl.pallas_call(
        _prep_body,
        grid=(GRID,),
        in_specs=[
            pl.BlockSpec((BN, in_dim), lambda i: (i, 0)),
            pl.BlockSpec((in_dim, F), lambda i: (0, 0)),
            pl.BlockSpec((F, 1), lambda i: (0, 0)),
            pl.BlockSpec((F, 1), lambda i: (0, 0)),
        ],
        out_specs=[
            pl.BlockSpec((BN, F), lambda i: (i, 0)),
            pl.BlockSpec((BN, 1), lambda i: (i, 0)),
            pl.BlockSpec((BN, 1), lambda i: (i, 0)),
        ],
        out_shape=[
            jax.ShapeDtypeStruct((N, F), jnp.float32),
            jax.ShapeDtypeStruct((N, 1), jnp.float32),
            jax.ShapeDtypeStruct((N, 1), jnp.float32),
        ],
    )(x, W, a_src.reshape(F, 1), a_dst.reshape(F, 1))


def _mid_body(acch_ref, accw_ref, h_ref, b_ref, w2_ref, as_ref, ad_ref,
              h2_out, asv_out, adv_out):
    num = acch_ref[0] + acch_ref[1] + h_ref[...]
    den = accw_ref[0] + accw_ref[1] + 1.0
    h1 = num / den + b_ref[...]
    z = jnp.where(h1 > 0.0, h1, jnp.exp(jnp.minimum(h1, 0.0)) - 1.0)
    h2 = jnp.dot(z, w2_ref[...], preferred_element_type=jnp.float32)
    a_s = jnp.dot(h2, as_ref[...])
    a_d = jnp.dot(h2, ad_ref[...])
    h2_out[...] = h2
    asv_out[...] = a_s
    adv_out[...] = a_d


def _tc_mid(acc_h, acc_w, h, b, W2, a_src, a_dst):
    return pl.pallas_call(
        _mid_body,
        grid=(GRID,),
        in_specs=[
            pl.BlockSpec((2, BN, F), lambda i: (0, i, 0)),
            pl.BlockSpec((2, BN, 1), lambda i: (0, i, 0)),
            pl.BlockSpec((BN, F), lambda i: (i, 0)),
            pl.BlockSpec((1, F), lambda i: (0, 0)),
            pl.BlockSpec((F, F), lambda i: (0, 0)),
            pl.BlockSpec((F, 1), lambda i: (0, 0)),
            pl.BlockSpec((F, 1), lambda i: (0, 0)),
        ],
        out_specs=[
            pl.BlockSpec((BN, F), lambda i: (i, 0)),
            pl.BlockSpec((BN, 1), lambda i: (i, 0)),
            pl.BlockSpec((BN, 1), lambda i: (i, 0)),
        ],
        out_shape=[
            jax.ShapeDtypeStruct((N, F), jnp.float32),
            jax.ShapeDtypeStruct((N, 1), jnp.float32),
            jax.ShapeDtypeStruct((N, 1), jnp.float32),
        ],
    )(acc_h, acc_w.reshape(2, N, 1), h, b.reshape(1, F), W2,
      a_src.reshape(F, 1), a_dst.reshape(F, 1))


def _fin_body(acch_ref, accw_ref, h_ref, b_ref, out_ref):
    num = acch_ref[0] + acch_ref[1] + h_ref[...]
    den = accw_ref[0] + accw_ref[1] + 1.0
    out_ref[...] = num / den + b_ref[...]


def _tc_fin(acc_h, acc_w, h, b):
    return pl.pallas_call(
        _fin_body,
        grid=(GRID,),
        in_specs=[
            pl.BlockSpec((2, BN, F), lambda i: (0, i, 0)),
            pl.BlockSpec((2, BN, 1), lambda i: (0, i, 0)),
            pl.BlockSpec((BN, F), lambda i: (i, 0)),
            pl.BlockSpec((1, F), lambda i: (0, 0)),
        ],
        out_specs=pl.BlockSpec((BN, F), lambda i: (i, 0)),
        out_shape=jax.ShapeDtypeStruct((N, F), jnp.float32),
    )(acc_h, acc_w.reshape(2, N, 1), h, b.reshape(1, F))


def _edge_body(edges_hbm, h_hbm, as_hbm, ad_hbm, zh_hbm, zw_hbm,
               acch_out, accw_out,
               idx0, idx1, asg0, asg1, adg0, adg1, asd0, asd1,
               hg0, hg1, hw0, hw1,
               stage_h, stage_s,
               acch_s, accw_s,
               semg0, semg1, sems0, sems1):
    c = lax.axis_index("c")
    s = lax.axis_index("s")
    wid = s * 2 + c
    idx = (idx0, idx1)
    asg = (asg0, asg1)
    adg = (adg0, adg1)
    asd = (asd0, asd1)
    hg = (hg0, hg1)
    hw = (hw0, hw1)
    semg = (semg0, semg1)
    sems = (sems0, sems1)

    nq = (NRC - s + 15) // 16
    pltpu.sync_copy(zh_hbm, stage_h)
    pltpu.sync_copy(zw_hbm, stage_s)

    def init_body(t, carry):
        r0 = (s + t * 16) * RC
        pltpu.sync_copy(stage_h, acch_s.at[pl.ds(r0, RC)])
        pltpu.sync_copy(stage_s, accw_s.at[pl.ds(r0, RC)])
        return carry

    lax.fori_loop(0, nq, init_body, 0)
    plsc.subcore_barrier()

    nk = (NSUP - wid + NTILES - 1) // NTILES

    def gather_list(b):
        out = []
        for j in range(G):
            sl = pl.ds(j * EB, EB)
            out.append((h_hbm.at[idx[b].at[j, 0]], hg[b].at[sl]))
            out.append((as_hbm.at[idx[b].at[j, 0]], asg[b].at[sl]))
            out.append((ad_hbm.at[idx[b].at[j, 1]], adg[b].at[sl]))
            out.append((as_hbm.at[idx[b].at[j, 1]], asd[b].at[sl]))
        return out

    def scatter_list(b):
        out = []
        for j in range(G):
            sl = pl.ds(j * EB, EB)
            out.append((hw[b].at[sl], acch_s.at[idx[b].at[j, 1]]))
            out.append((asg[b].at[sl], accw_s.at[idx[b].at[j, 1]]))
        return out

    def fire_gathers(b, t):
        r0 = (wid + t * NTILES) * G
        pltpu.sync_copy(edges_hbm.at[pl.ds(r0, G)], idx[b])
        for a, v in gather_list(b):
            pltpu.async_copy(a, v, semg[b])

    def drain_gathers(b):
        for a, v in gather_list(b):
            pltpu.make_async_copy(a, v, semg[b]).wait()

    def fire_scatters(b):
        for v, a in scatter_list(b):
            pltpu.async_copy(v, a, sems[b], add=True)

    def drain_scatters(b):
        for v, a in scatter_list(b):
            pltpu.make_async_copy(v, a, sems[b]).wait()

    def compute(b):
        def blk(j, carry):
            sl = pl.ds(j * 16, 16)
            ad16 = adg[b][sl]
            t0 = asg[b][sl] + ad16
            e = jnp.maximum(t0, 0.2 * t0)
            td = asd[b][sl] + ad16
            cg = jnp.maximum(td, 0.2 * td)
            w = jnp.exp(e - cg)
            asg[b][sl] = w
            for k in range(16):
                i = j * 16 + k
                hw[b][i, :] = hg[b][i, :] * w[k]
            return carry

        lax.fori_loop(0, G * EB // 16, blk, 0)

    def phase(b, t):
        drain_gathers(b)

        @pl.when(t >= 1)
        def _():
            drain_scatters(1 - b)

        @pl.when(t + 1 < nk)
        def _():
            fire_gathers(1 - b, t + 1)

        compute(b)
        fire_scatters(b)

    fire_gathers(0, 0)

    def loop_body(t, carry):
        @pl.when(t % 2 == 0)
        def _():
            phase(0, t)

        @pl.when(t % 2 == 1)
        def _():
            phase(1, t)

        return carry

    lax.fori_loop(0, nk, loop_body, 0)

    @pl.when((nk - 1) % 2 == 0)
    def _():
        drain_scatters(0)

    @pl.when((nk - 1) % 2 == 1)
    def _():
        drain_scatters(1)

    plsc.subcore_barrier()

    def wb_body(t, carry):
        r0 = (s + t * 16) * RC
        pltpu.sync_copy(acch_s.at[pl.ds(r0, RC)], stage_h)
        pltpu.sync_copy(stage_h, acch_out.at[c, pl.ds(r0, RC)])
        pltpu.sync_copy(accw_s.at[pl.ds(r0, RC)], stage_s)
        pltpu.sync_copy(stage_s, accw_out.at[pl.ds(c * N + r0, RC)])
        return carry

    lax.fori_loop(0, nq, wb_body, 0)


def _edge_pass(edges, h, asv, adv, zh, zw):
    mesh = plsc.VectorSubcoreMesh(core_axis_name="c", subcore_axis_name="s")
    f = functools.partial(
        pl.kernel,
        mesh=mesh,
        compiler_params=pltpu.CompilerParams(use_tc_tiling_on_sc=False),
        out_type=[
            jax.ShapeDtypeStruct((2, N, F), jnp.float32),
            jax.ShapeDtypeStruct((2 * N,), jnp.float32),
        ],
        scratch_types=[
            pltpu.VMEM((G, 2, EB), jnp.int32),
            pltpu.VMEM((G, 2, EB), jnp.int32),
            pltpu.VMEM((G * EB,), jnp.float32),
            pltpu.VMEM((G * EB,), jnp.float32),
            pltpu.VMEM((G * EB,), jnp.float32),
            pltpu.VMEM((G * EB,), jnp.float32),
            pltpu.VMEM((G * EB,), jnp.float32),
            pltpu.VMEM((G * EB,), jnp.float32),
            pltpu.VMEM((G * EB, F), jnp.float32),
            pltpu.VMEM((G * EB, F), jnp.float32),
            pltpu.VMEM((G * EB, F), jnp.float32),
            pltpu.VMEM((G * EB, F), jnp.float32),
            pltpu.VMEM((RC, F), jnp.float32),
            pltpu.VMEM((RC,), jnp.float32),
            pltpu.VMEM_SHARED((N, F), jnp.float32),
            pltpu.VMEM_SHARED((N,), jnp.float32),
            pltpu.SemaphoreType.DMA,
            pltpu.SemaphoreType.DMA,
            pltpu.SemaphoreType.DMA,
            pltpu.SemaphoreType.DMA,
        ],
    )(_edge_body)
    return f(edges, h, asv, adv, zh, zw)


def kernel(x, edge_index, W1, a1_src, a1_dst, b1, W2, a2_src, a2_dst, b2):
    edges = jnp.stack([edge_index[0].astype(jnp.int32).reshape(NCHUNK, EB),
                       edge_index[1].astype(jnp.int32).reshape(NCHUNK, EB)],
                      axis=1)
    zh = jnp.zeros((RC, F), jnp.float32)
    zw = jnp.zeros((RC,), jnp.float32)

    h1, as1, ad1 = _tc_prep(x, W1, a1_src, a1_dst)
    acch1, accw1 = _edge_pass(edges, h1, as1.reshape(N), ad1.reshape(N),
                              zh, zw)
    h2, as2, ad2 = _tc_mid(acch1, accw1, h1, b1, W2, a2_src, a2_dst)
    acch2, accw2 = _edge_pass(edges, h2, as2.reshape(N), ad2.reshape(N),
                              zh, zw)
    return _tc_fin(acch2, accw2, h2, b2)


# speedup vs baseline: 91.8205x; 1.0126x over previous
import functools

import jax
import jax.numpy as jnp
from jax import lax
from jax.experimental import pallas as pl
from jax.experimental.pallas import tpu as pltpu
from jax.experimental.pallas import tpu_sc as plsc

N = 50000
E = 1600000
F = 16
BN = 2000
GRID = N // BN
EB = 128
NCHUNK = E // EB
G = 5
NSUP = NCHUNK // G
RC = 200
NRC = N // RC
NTILES = 32


def _prep_body(x_ref, w_ref, as_ref, ad_ref, h_out, asv_out, adv_out):
    h = jnp.dot(x_ref[...], w_ref[...], preferred_element_type=jnp.float32)
    h_out[...] = h
    asv_out[...] = jnp.dot(h, as_ref[...])
    adv_out[...] = jnp.dot(h, ad_ref[...])


def _tc_prep(x, W, a_src, a_dst):
    in_dim = x.shape[1]
    return pl.pallas_call(
        _prep_body,
        grid=(GRID,),
        in_specs=[
            pl.BlockSpec((BN, in_dim), lambda i: (i, 0)),
            pl.BlockSpec((in_dim, F), lambda i: (0, 0)),
            pl.BlockSpec((F, 1), lambda i: (0, 0)),
            pl.BlockSpec((F, 1), lambda i: (0, 0)),
        ],
        out_specs=[
            pl.BlockSpec((BN, F), lambda i: (i, 0)),
            pl.BlockSpec((BN, 1), lambda i: (i, 0)),
            pl.BlockSpec((BN, 1), lambda i: (i, 0)),
        ],
        out_shape=[
            jax.ShapeDtypeStruct((N, F), jnp.float32),
            jax.ShapeDtypeStruct((N, 1), jnp.float32),
            jax.ShapeDtypeStruct((N, 1), jnp.float32),
        ],
    )(x, W, a_src.reshape(F, 1), a_dst.reshape(F, 1))


def _mid_body(acch_ref, accw_ref, h_ref, b_ref, w2_ref, as_ref, ad_ref,
              h2_out, asv_out, adv_out):
    num = acch_ref[0] + acch_ref[1] + h_ref[...]
    den = accw_ref[0] + accw_ref[1] + 1.0
    h1 = num / den + b_ref[...]
    z = jnp.where(h1 > 0.0, h1, jnp.exp(jnp.minimum(h1, 0.0)) - 1.0)
    h2 = jnp.dot(z, w2_ref[...], preferred_element_type=jnp.float32)
    a_s = jnp.dot(h2, as_ref[...])
    a_d = jnp.dot(h2, ad_ref[...])
    h2_out[...] = h2
    asv_out[...] = a_s
    adv_out[...] = a_d


def _tc_mid(acc_h, acc_w, h, b, W2, a_src, a_dst):
    return pl.pallas_call(
        _mid_body,
        grid=(GRID,),
        in_specs=[
            pl.BlockSpec((2, BN, F), lambda i: (0, i, 0)),
            pl.BlockSpec((2, BN, 1), lambda i: (0, i, 0)),
            pl.BlockSpec((BN, F), lambda i: (i, 0)),
            pl.BlockSpec((1, F), lambda i: (0, 0)),
            pl.BlockSpec((F, F), lambda i: (0, 0)),
            pl.BlockSpec((F, 1), lambda i: (0, 0)),
            pl.BlockSpec((F, 1), lambda i: (0, 0)),
        ],
        out_specs=[
            pl.BlockSpec((BN, F), lambda i: (i, 0)),
            pl.BlockSpec((BN, 1), lambda i: (i, 0)),
            pl.BlockSpec((BN, 1), lambda i: (i, 0)),
        ],
        out_shape=[
            jax.ShapeDtypeStruct((N, F), jnp.float32),
            jax.ShapeDtypeStruct((N, 1), jnp.float32),
            jax.ShapeDtypeStruct((N, 1), jnp.float32),
        ],
    )(acc_h, acc_w.reshape(2, N, 1), h, b.reshape(1, F), W2,
      a_src.reshape(F, 1), a_dst.reshape(F, 1))


def _fin_body(acch_ref, accw_ref, h_ref, b_ref, out_ref):
    num = acch_ref[0] + acch_ref[1] + h_ref[...]
    den = accw_ref[0] + accw_ref[1] + 1.0
    out_ref[...] = num / den + b_ref[...]


def _tc_fin(acc_h, acc_w, h, b):
    return pl.pallas_call(
        _fin_body,
        grid=(GRID,),
        in_specs=[
            pl.BlockSpec((2, BN, F), lambda i: (0, i, 0)),
            pl.BlockSpec((2, BN, 1), lambda i: (0, i, 0)),
            pl.BlockSpec((BN, F), lambda i: (i, 0)),
            pl.BlockSpec((1, F), lambda i: (0, 0)),
        ],
        out_specs=pl.BlockSpec((BN, F), lambda i: (i, 0)),
        out_shape=jax.ShapeDtypeStruct((N, F), jnp.float32),
    )(acc_h, acc_w.reshape(2, N, 1), h, b.reshape(1, F))


def _edge_body(edges_hbm, h_hbm, as_hbm, ad_hbm, zh_hbm, zw_hbm,
               acch_out, accw_out,
               idx0, idx1, asg0, asg1, adg0, adg1, asd0, asd1,
               hg0, hg1, hw0, hw1,
               stage_h, stage_s,
               acch_s, accw_s,
               semg0, semg1, sems0, sems1):
    c = lax.axis_index("c")
    s = lax.axis_index("s")
    wid = s * 2 + c
    idx = (idx0, idx1)
    asg = (asg0, asg1)
    adg = (adg0, adg1)
    asd = (asd0, asd1)
    hg = (hg0, hg1)
    hw = (hw0, hw1)
    semg = (semg0, semg1)
    sems = (sems0, sems1)

    nq = (NRC - s + 15) // 16
    pltpu.sync_copy(zh_hbm, stage_h)
    pltpu.sync_copy(zw_hbm, stage_s)

    def init_body(t, carry):
        r0 = (s + t * 16) * RC
        pltpu.sync_copy(stage_h, acch_s.at[pl.ds(r0, RC)])
        pltpu.sync_copy(stage_s, accw_s.at[pl.ds(r0, RC)])
        return carry

    lax.fori_loop(0, nq, init_body, 0)
    plsc.subcore_barrier()

    nk = (NSUP - wid + NTILES - 1) // NTILES

    def gather_list(b):
        out = []
        for j in range(G):
            sl = pl.ds(j * EB, EB)
            out.append((h_hbm.at[idx[b].at[j, 0]], hg[b].at[sl]))
            out.append((as_hbm.at[idx[b].at[j, 0]], asg[b].at[sl]))
            out.append((ad_hbm.at[idx[b].at[j, 1]], adg[b].at[sl]))
            out.append((as_hbm.at[idx[b].at[j, 1]], asd[b].at[sl]))
        return out

    def scatter_list(b):
        out = []
        for j in range(G):
            sl = pl.ds(j * EB, EB)
            out.append((hw[b].at[sl], acch_s.at[idx[b].at[j, 1]]))
            out.append((asg[b].at[sl], accw_s.at[idx[b].at[j, 1]]))
        return out

    def fire_gathers(b, t):
        r0 = (wid + t * NTILES) * G
        pltpu.sync_copy(edges_hbm.at[pl.ds(r0, G)], idx[b])
        for a, v in gather_list(b):
            pltpu.async_copy(a, v, semg[b])

    def drain_gathers(b):
        for a, v in gather_list(b):
            pltpu.make_async_copy(a, v, semg[b]).wait()

    def fire_scatters(b):
        for v, a in scatter_list(b):
            pltpu.async_copy(v, a, sems[b], add=True)

    def drain_scatters(b):
        for v, a in scatter_list(b):
            pltpu.make_async_copy(v, a, sems[b]).wait()

    def compute(b):
        def blk(j, carry):
            sl = pl.ds(j * 16, 16)
            ad16 = adg[b][sl]
            t0 = asg[b][sl] + ad16
            e = jnp.maximum(t0, 0.2 * t0)
            td = asd[b][sl] + ad16
            cg = jnp.maximum(td, 0.2 * td)
            w = jnp.exp(e - cg)
            asg[b][sl] = w
            for k in range(16):
                i = j * 16 + k
                hw[b][i, :] = hg[b][i, :] * w[k]
            return carry

        lax.fori_loop(0, G * EB // 16, blk, 0)

    def phase(b, t):
        drain_gathers(b)

        @pl.when(t >= 1)
        def _():
            drain_scatters(1 - b)

        @pl.when(t + 1 < nk)
        def _():
            fire_gathers(1 - b, t + 1)

        compute(b)
        fire_scatters(b)

    fire_gathers(0, 0)

    def loop_body(t, carry):
        @pl.when(t % 2 == 0)
        def _():
            phase(0, t)

        @pl.when(t % 2 == 1)
        def _():
            phase(1, t)

        return carry

    lax.fori_loop(0, nk, loop_body, 0)

    @pl.when((nk - 1) % 2 == 0)
    def _():
        drain_scatters(0)

    @pl.when((nk - 1) % 2 == 1)
    def _():
        drain_scatters(1)

    plsc.subcore_barrier()

    def wb_body(t, carry):
        r0 = (s + t * 16) * RC
        pltpu.sync_copy(acch_s.at[pl.ds(r0, RC)], stage_h)
        pltpu.sync_copy(stage_h, acch_out.at[c, pl.ds(r0, RC)])
        pltpu.sync_copy(accw_s.at[pl.ds(r0, RC)], stage_s)
        pltpu.sync_copy(stage_s, accw_out.at[pl.ds(c * N + r0, RC)])
        return carry

    lax.fori_loop(0, nq, wb_body, 0)


def _edge_pass(edges, h, asv, adv, zh, zw):
    mesh = plsc.VectorSubcoreMesh(core_axis_name="c", subcore_axis_name="s")
    f = functools.partial(
        pl.kernel,
        mesh=mesh,
        compiler_params=pltpu.CompilerParams(use_tc_tiling_on_sc=False),
        out_type=[
            jax.ShapeDtypeStruct((2, N, F), jnp.float32),
            jax.ShapeDtypeStruct((2 * N,), jnp.float32),
        ],
        scratch_types=[
            pltpu.VMEM((G, 2, EB), jnp.int32),
            pltpu.VMEM((G, 2, EB), jnp.int32),
            pltpu.VMEM((G * EB,), jnp.float32),
            pltpu.VMEM((G * EB,), jnp.float32),
            pltpu.VMEM((G * EB,), jnp.float32),
            pltpu.VMEM((G * EB,), jnp.float32),
            pltpu.VMEM((G * EB,), jnp.float32),
            pltpu.VMEM((G * EB,), jnp.float32),
            pltpu.VMEM((G * EB, F), jnp.float32),
            pltpu.VMEM((G * EB, F), jnp.float32),
            pltpu.VMEM((G * EB, F), jnp.float32),
            pltpu.VMEM((G * EB, F), jnp.float32),
            pltpu.VMEM((RC, F), jnp.float32),
            pltpu.VMEM((RC,), jnp.float32),
            pltpu.VMEM_SHARED((N, F), jnp.float32),
            pltpu.VMEM_SHARED((N,), jnp.float32),
            pltpu.SemaphoreType.DMA,
            pltpu.SemaphoreType.DMA,
            pltpu.SemaphoreType.DMA,
            pltpu.SemaphoreType.DMA,
        ],
    )(_edge_body)
    return f(edges, h, asv, adv, zh, zw)


def kernel(x, edge_index, W1, a1_src, a1_dst, b1, W2, a2_src, a2_dst, b2):
    edges = jnp.stack([edge_index[0].astype(jnp.int32).reshape(NCHUNK, EB),
                       edge_index[1].astype(jnp.int32).reshape(NCHUNK, EB)],
                      axis=1)
    zh = jnp.zeros((RC, F), jnp.float32)
    zw = jnp.zeros((RC,), jnp.float32)

    h1, as1, ad1 = _tc_prep(x, W1, a1_src, a1_dst)
    acch1, accw1 = _edge_pass(edges, h1, as1.reshape(N), ad1.reshape(N),
                              zh, zw)
    h2, as2, ad2 = _tc_mid(acch1, accw1, h1, b1, W2, a2_src, a2_dst)
    acch2, accw2 = _edge_pass(edges, h2, as2.reshape(N), ad2.reshape(N),
                              zh, zw)
    return _tc_fin(acch2, accw2, h2, b2)
